# baseline (speedup 1.0000x reference)
import jax
import jax.numpy as jnp
from jax import lax
from jax.experimental import pallas as pl
from jax.experimental.pallas import tpu as pltpu

_N = 8192
_D = 4096
_E = 64
_R = 16
_C = _E * _R
_BT = 512


def _body(x_ref, sh_ref, w_ref, sel_ref):
    x = x_ref[...]
    nrm = jnp.sqrt(jnp.sum(x * x, axis=1, keepdims=True))
    xn = x * (1.0 / jnp.maximum(nrm, 1e-12))

    xh = xn.astype(jnp.bfloat16)
    sh = sh_ref[...]
    proj = jnp.dot(xh, sh, preferred_element_type=jnp.float32)

    p2 = proj * proj
    acc = p2[:, 0:128]
    for k in range(1, 8):
        acc = acc + p2[:, k * 128:(k + 1) * 128]
    o2 = acc[:, 0:64] + acc[:, 64:128]

    logits = jnp.sqrt(o2) * -10.0
    m = jnp.max(logits, axis=1, keepdims=True)
    e = jnp.exp(logits - m)
    w = e / jnp.sum(e, axis=1, keepdims=True)
    w_ref[...] = w

    iota = lax.broadcasted_iota(jnp.int32, (_BT, _E), 1)
    m1 = jnp.max(w, axis=1, keepdims=True)
    i1 = jnp.min(jnp.where(w == m1, iota, _E), axis=1, keepdims=True)
    w2 = jnp.where(iota == i1, -1.0, w)
    m2 = jnp.max(w2, axis=1, keepdims=True)
    i2 = jnp.min(jnp.where(w2 == m2, iota, _E), axis=1, keepdims=True)
    sel_ref[...] = jnp.concatenate([i1, i2], axis=1)


def _route(x, sh):
    n = x.shape[0]
    grid = (n // _BT,)
    return pl.pallas_call(
        _body,
        grid=grid,
        in_specs=[
            pl.BlockSpec((_BT, _D), lambda i: (i, 0)),
            pl.BlockSpec((_D, _C), lambda i: (0, 0)),
        ],
        out_specs=[
            pl.BlockSpec((_BT, _E), lambda i: (i, 0)),
            pl.BlockSpec((_BT, 2), lambda i: (i, 0)),
        ],
        out_shape=[
            jax.ShapeDtypeStruct((n, _E), jnp.float32),
            jax.ShapeDtypeStruct((n, 2), jnp.int32),
        ],
        compiler_params=pltpu.CompilerParams(
            dimension_semantics=("parallel",),
        ),
    )(x, sh)


def kernel(x, expert_subspaces):
    s = expert_subspaces.transpose(1, 2, 0).reshape(_D, _C)
    sh = s.astype(jnp.bfloat16)

    devs = jax.devices()
    if len(devs) >= 2:
        mesh = jax.sharding.Mesh(devs[:2], ("dp",))
        P = jax.sharding.PartitionSpec
        f = jax.shard_map(
            _route,
            mesh=mesh,
            in_specs=(P("dp", None), P(None, None)),
            out_specs=(P("dp", None), P("dp", None)),
            check_vma=False,
        )
        return f(x, sh)
    return _route(x, sh)


# speedup vs baseline: 4.2790x; 4.2790x over previous
import jax
import jax.numpy as jnp
from jax import lax
from jax.experimental import pallas as pl
from jax.experimental.pallas import tpu as pltpu

_N = 8192
_D = 4096
_E = 64
_R = 16
_C = _E * _R
_BT = 1024


_CH = 256


def _body(x_ref, sh_ref, w_ref, sel_ref):
    sh = sh_ref[...]
    for c in range(_BT // _CH):
        sl = pl.ds(c * _CH, _CH)
        x = x_ref[sl, :]
        nrm = jnp.sqrt(jnp.sum(x * x, axis=1, keepdims=True))
        xn = x * (1.0 / jnp.maximum(nrm, 1e-12))

        xh = xn.astype(jnp.bfloat16)
        proj = jnp.dot(xh, sh, preferred_element_type=jnp.float32)

        p2 = proj * proj
        acc = p2[:, 0:128]
        for k in range(1, 8):
            acc = acc + p2[:, k * 128:(k + 1) * 128]
        o2 = acc[:, 0:64] + acc[:, 64:128]

        logits = jnp.sqrt(o2) * -10.0
        m = jnp.max(logits, axis=1, keepdims=True)
        e = jnp.exp(logits - m)
        w = e / jnp.sum(e, axis=1, keepdims=True)
        w_ref[sl, :] = w

        iota = lax.broadcasted_iota(jnp.int32, (_CH, _E), 1)
        m1 = jnp.max(w, axis=1, keepdims=True)
        i1 = jnp.min(jnp.where(w == m1, iota, _E), axis=1, keepdims=True)
        w2 = jnp.where(iota == i1, -1.0, w)
        m2 = jnp.max(w2, axis=1, keepdims=True)
        i2 = jnp.min(jnp.where(w2 == m2, iota, _E), axis=1, keepdims=True)
        sel_ref[sl, :] = jnp.concatenate([i1, i2], axis=1)


def _route(x, sh):
    n = x.shape[0]
    grid = (n // _BT,)
    return pl.pallas_call(
        _body,
        grid=grid,
        in_specs=[
            pl.BlockSpec((_BT, _D), lambda i: (i, 0)),
            pl.BlockSpec((_D, _C), lambda i: (0, 0)),
        ],
        out_specs=[
            pl.BlockSpec((_BT, _E), lambda i: (i, 0)),
            pl.BlockSpec((_BT, 2), lambda i: (i, 0)),
        ],
        out_shape=[
            jax.ShapeDtypeStruct((n, _E), jnp.float32),
            jax.ShapeDtypeStruct((n, 2), jnp.int32),
        ],
        compiler_params=pltpu.CompilerParams(
            dimension_semantics=("parallel",),
        ),
    )(x, sh)


def kernel(x, expert_subspaces):
    s = expert_subspaces.transpose(1, 2, 0).reshape(_D, _C)
    sh = s.astype(jnp.bfloat16)

    return _route(x, sh)
